# Initial kernel scaffold; baseline (speedup 1.0000x reference)
import jax
import jax.numpy as jnp
from jax.experimental import pallas as pl


def kernel(ef_values, ef_rows, ef_cols, dense_weight_slice):
    raise NotImplementedError("write your pallas kernel here")


# speedup vs baseline: 4.8422x; 4.8422x over previous
import functools

import jax
import jax.numpy as jnp
from jax import lax
from jax.experimental import pallas as pl
from jax.experimental.pallas import tpu as pltpu
from jax.experimental.pallas import tpu_sc as plsc

N_USERS = 65536
N_ITEMS = 65536
NNZ = 2097152
D = 64

NC = 2
NS = 16
NW = NC * NS
NB = 64
RPB = N_USERS // NB
BPW = NB // NW
E = 128
ACC_ROWS = RPB + 32


def _body(vals_hbm, rows_hbm, cols_hbm, w_hbm, bnd_hbm, out_hbm,
          bounds_v, cols_v, rows_v, vals_v, gbuf, acc, sem):
    wid = lax.axis_index("s") * NC + lax.axis_index("c")
    pltpu.sync_copy(bnd_hbm, bounds_v)

    for j in range(BPW):
        b = wid * BPW + j
        base = b * RPB

        def zero_body(r):
            for c in range(D // 16):
                acc[r, pl.ds(c * 16, 16)] = jnp.zeros((16,), jnp.float32)

        plsc.parallel_loop(0, ACC_ROWS, 1, unroll=8)(zero_body)

        s = bounds_v[pl.ds(b, 16)][0]
        t = bounds_v[pl.ds(b + 1, 16)][0]
        k0 = s // E
        k1 = (t + E - 1) // E

        def chunk_body(k, _):
            e0 = k * E
            pltpu.sync_copy(cols_hbm.at[pl.ds(e0, E)], cols_v)
            pltpu.sync_copy(rows_hbm.at[pl.ds(e0, E)], rows_v.at[pl.ds(0, E)])
            pltpu.sync_copy(vals_hbm.at[pl.ds(e0, E)], vals_v)
            pltpu.async_copy(w_hbm.at[cols_v], gbuf, sem).wait()

            def edge_body(e, _):
                r = rows_v[pl.ds(e, 16)][0]
                off = jnp.where((r >= base) & (r < base + RPB), r - base, RPB)
                val = plsc.load_gather(vals_v, [jnp.full((16,), e, jnp.int32)])
                for c in range(D // 16):
                    g = gbuf[e, pl.ds(c * 16, 16)]
                    plsc.addupdate(acc.at[off, pl.ds(c * 16, 16)], g * val)
                return _

            lax.fori_loop(0, E, edge_body, None)
            return _

        lax.fori_loop(k0, k1, chunk_body, None)
        pltpu.sync_copy(acc.at[pl.ds(0, RPB)], out_hbm.at[pl.ds(base, RPB)])


def kernel(ef_values, ef_rows, ef_cols, dense_weight_slice):
    edges = jnp.arange(0, N_USERS + 1, RPB, dtype=jnp.int32)
    bounds = jnp.searchsorted(ef_rows, edges, side="left").astype(jnp.int32)
    bnd = jnp.zeros((128,), jnp.int32).at[: NB + 1].set(bounds)

    mesh = plsc.VectorSubcoreMesh(core_axis_name="c", subcore_axis_name="s")
    run = pl.kernel(
        _body,
        out_type=jax.ShapeDtypeStruct((N_USERS, D), jnp.float32),
        mesh=mesh,
        compiler_params=pltpu.CompilerParams(
            needs_layout_passes=False, use_tc_tiling_on_sc=False
        ),
        scratch_types=[
            pltpu.VMEM((128,), jnp.int32),
            pltpu.VMEM((E,), jnp.int32),
            pltpu.VMEM((E + 16,), jnp.int32),
            pltpu.VMEM((E,), jnp.float32),
            pltpu.VMEM((E, D), jnp.float32),
            pltpu.VMEM((ACC_ROWS, D), jnp.float32),
            pltpu.SemaphoreType.DMA,
        ],
    )
    return run(ef_values, ef_rows, ef_cols, dense_weight_slice, bnd)


# speedup vs baseline: 6.3278x; 1.3068x over previous
import jax
import jax.numpy as jnp
from jax import lax
from jax.experimental import pallas as pl
from jax.experimental.pallas import tpu as pltpu
from jax.experimental.pallas import tpu_sc as plsc

N_USERS = 65536
N_ITEMS = 65536
NNZ = 2097152
D = 64

NC = 2
NS = 16
NW = NC * NS
NB = 128
RPB = N_USERS // NB
BPW = NB // NW
E = 512
ACC_ROWS = RPB + 32
BND = 160


def _body(vals_hbm, rows_hbm, cols_hbm, w_hbm, bnd_hbm, out_hbm,
          bounds_v, cols_v, rows_v, vals_v, gbuf, acc,
          sem_g, sem_c, sem_r, sem_v):
    wid = lax.axis_index("s") * NC + lax.axis_index("c")
    pltpu.sync_copy(bnd_hbm, bounds_v)

    for j in range(BPW):
        b = wid * BPW + j
        base = b * RPB

        def zero_body(r):
            for c in range(D // 16):
                acc[r, pl.ds(c * 16, 16)] = jnp.zeros((16,), jnp.float32)

        plsc.parallel_loop(0, ACC_ROWS, 1, unroll=8)(zero_body)

        s = bounds_v[pl.ds(b, 16)][0]
        t = bounds_v[pl.ds(b + 1, 16)][0]
        k0 = s // E
        k1 = (t + E - 1) // E

        def chunk_body(k, _):
            e0 = k * E
            cp_c = pltpu.async_copy(cols_hbm.at[pl.ds(e0, E)], cols_v, sem_c)
            cp_r = pltpu.async_copy(rows_hbm.at[pl.ds(e0, E)],
                                    rows_v.at[pl.ds(0, E)], sem_r)
            cp_v = pltpu.async_copy(vals_hbm.at[pl.ds(e0, E)], vals_v, sem_v)
            cp_c.wait()
            cp_g = pltpu.async_copy(w_hbm.at[cols_v], gbuf, sem_g)
            cp_r.wait()
            cp_v.wait()
            cp_g.wait()

            def edge_body(e, _):
                r = rows_v[pl.ds(e, 16)][0]
                off = jnp.where((r >= base) & (r < base + RPB), r - base, RPB)
                val = plsc.load_gather(vals_v, [jnp.full((16,), e, jnp.int32)])
                for c in range(D // 16):
                    g = gbuf[e, pl.ds(c * 16, 16)]
                    plsc.addupdate(acc.at[off, pl.ds(c * 16, 16)], g * val)
                return _

            lax.fori_loop(0, E, edge_body, None)
            return _

        lax.fori_loop(k0, k1, chunk_body, None)
        pltpu.sync_copy(acc.at[pl.ds(0, RPB)], out_hbm.at[pl.ds(base, RPB)])


def kernel(ef_values, ef_rows, ef_cols, dense_weight_slice):
    edges = jnp.arange(0, N_USERS + 1, RPB, dtype=jnp.int32)
    bounds = jnp.searchsorted(ef_rows, edges, side="left").astype(jnp.int32)
    bnd = jnp.zeros((BND,), jnp.int32).at[: NB + 1].set(bounds)

    mesh = plsc.VectorSubcoreMesh(core_axis_name="c", subcore_axis_name="s")
    run = pl.kernel(
        _body,
        out_type=jax.ShapeDtypeStruct((N_USERS, D), jnp.float32),
        mesh=mesh,
        compiler_params=pltpu.CompilerParams(
            needs_layout_passes=False, use_tc_tiling_on_sc=False
        ),
        scratch_types=[
            pltpu.VMEM((BND,), jnp.int32),
            pltpu.VMEM((E,), jnp.int32),
            pltpu.VMEM((E + 16,), jnp.int32),
            pltpu.VMEM((E,), jnp.float32),
            pltpu.VMEM((E, D), jnp.float32),
            pltpu.VMEM((ACC_ROWS, D), jnp.float32),
            pltpu.SemaphoreType.DMA,
            pltpu.SemaphoreType.DMA,
            pltpu.SemaphoreType.DMA,
            pltpu.SemaphoreType.DMA,
        ],
    )
    return run(ef_values, ef_rows, ef_cols, dense_weight_slice, bnd)
